# baseline (speedup 1.0000x reference)
import jax
import jax.numpy as jnp
from jax import lax
from jax.experimental import pallas as pl

_E = 16


def _mlp_body(x_ref, skill_ref, w1_ref, b1_ref, w2_ref, b2_ref, w3e_ref,
              b3_ref, out_ref):
    x = x_ref[...]
    acc = jnp.zeros((x.shape[0], _E), jnp.float32)
    for e in range(_E):
        h = jnp.dot(x, w1_ref[e], preferred_element_type=jnp.float32)
        h = jnp.maximum(h + b1_ref[e], 0.0).astype(jnp.bfloat16)
        h = jnp.dot(h, w2_ref[e], preferred_element_type=jnp.float32)
        h = jnp.maximum(h + b2_ref[e], 0.0).astype(jnp.bfloat16)
        acc = acc + jnp.dot(h, w3e_ref[e], preferred_element_type=jnp.float32)

    onehot = (skill_ref[...] ==
              lax.broadcasted_iota(jnp.int32, acc.shape, 1)).astype(jnp.float32)
    logit = jnp.sum((acc + b3_ref[...]) * onehot, axis=1, keepdims=True)
    out_ref[...] = jax.nn.sigmoid(logit)


def kernel(observation, action, skill_idx, W1, b1, W2, b2, W3, b3):
    batch = observation.shape[0]
    h1 = W1.shape[2]
    h2 = W2.shape[2]

    x = jnp.concatenate([observation, action], axis=1).astype(jnp.bfloat16)
    skill = skill_idx.astype(jnp.int32).reshape(batch, 1)
    b1r = b1.reshape(_E, 1, h1)
    b2r = b2.reshape(_E, 1, h2)
    w3e = (W3.reshape(_E, h2, 1) *
           jnp.eye(_E, dtype=jnp.float32)[:, None, :]).astype(jnp.bfloat16)
    b3r = b3.reshape(1, _E)

    out = pl.pallas_call(
        _mlp_body,
        in_specs=[
            pl.BlockSpec((batch, x.shape[1]), lambda: (0, 0)),
            pl.BlockSpec((batch, 1), lambda: (0, 0)),
            pl.BlockSpec((_E, x.shape[1], h1), lambda: (0, 0, 0)),
            pl.BlockSpec((_E, 1, h1), lambda: (0, 0, 0)),
            pl.BlockSpec((_E, h1, h2), lambda: (0, 0, 0)),
            pl.BlockSpec((_E, 1, h2), lambda: (0, 0, 0)),
            pl.BlockSpec((_E, h2, _E), lambda: (0, 0, 0)),
            pl.BlockSpec((1, _E), lambda: (0, 0)),
        ],
        out_specs=pl.BlockSpec((batch, 1), lambda: (0, 0)),
        out_shape=jax.ShapeDtypeStruct((batch, 1), jnp.float32),
    )(x, skill, W1.astype(jnp.bfloat16), b1r, W2.astype(jnp.bfloat16),
      b2r, w3e, b3r)
    return out
